# baseline (speedup 1.0000x reference)
import functools

import jax
import jax.numpy as jnp
from jax import lax
from jax.experimental import pallas as pl
from jax.experimental.pallas import tpu as pltpu
from jax.experimental.pallas import tpu_sc as plsc

N, D, S = 50000, 512, 64
NC, NS = 2, 16
NW = NC * NS
CH = 80
NB = CH // 16
TB = 20
TS = 19
BIGW = 17
RB = CH * TB
RS = CH * TS
DV = D // 16
CW = 16
CHD = CH * D


def _sc_segment_sums(xf, batch):
    mesh = plsc.VectorSubcoreMesh(core_axis_name="c", subcore_axis_name="s")

    @functools.partial(
        pl.kernel,
        mesh=mesh,
        out_type=[
            jax.ShapeDtypeStruct((NW, S * D), jnp.float32),
            jax.ShapeDtypeStruct((NW, S * CW), jnp.float32),
        ],
        scratch_types=[
            pltpu.VMEM((2, CH, D), jnp.float32),
            pltpu.VMEM((RB + 16,), jnp.int32),
            pltpu.VMEM((S * D,), jnp.float32),
            pltpu.VMEM((S * CW,), jnp.float32),
            pltpu.SemaphoreType.DMA,
            pltpu.SemaphoreType.DMA,
        ],
    )
    def seg(x_hbm, b_hbm, sums_hbm, cnts_hbm, rows, idxv, acc, cntv,
            sem0, sem1):
        cid = lax.axis_index("c")
        sid = lax.axis_index("s")
        wid = cid * NS + sid
        big = wid < BIGW
        nch = jnp.where(big, TB, TS)
        wbase = jnp.where(big, wid * RB, BIGW * RB + (wid - BIGW) * RS)

        zv = jnp.zeros((16,), jnp.float32)

        def zero_body(s, _):
            for j in range(DV):
                acc[pl.ds(s * D + 16 * j, 16)] = zv
            return 0

        lax.fori_loop(0, S, zero_body, 0)
        for k in range(S * CW // 16):
            cntv[pl.ds(16 * k, 16)] = zv

        @pl.when(big)
        def _():
            pltpu.sync_copy(b_hbm.at[pl.ds(wbase, RB)], idxv.at[pl.ds(0, RB)])

        @pl.when(jnp.logical_not(big))
        def _():
            pltpu.sync_copy(b_hbm.at[pl.ds(wbase, RS)], idxv.at[pl.ds(0, RS)])

        def issue(t):
            @pl.when(lax.rem(t, 2) == 0)
            def _():
                pltpu.async_copy(
                    x_hbm.at[pl.ds(wbase + t * CH, CH)],
                    rows.at[0], sem0)

            @pl.when(lax.rem(t, 2) == 1)
            def _():
                pltpu.async_copy(
                    x_hbm.at[pl.ds(wbase + t * CH, CH)],
                    rows.at[1], sem1)

        def wait_t(t):
            @pl.when(lax.rem(t, 2) == 0)
            def _():
                pltpu.make_async_copy(
                    x_hbm.at[pl.ds(wbase + t * CH, CH)],
                    rows.at[0], sem0).wait()

            @pl.when(lax.rem(t, 2) == 1)
            def _():
                pltpu.make_async_copy(
                    x_hbm.at[pl.ds(wbase + t * CH, CH)],
                    rows.at[1], sem1).wait()

        sixteen = jnp.full((CW,), 16.0, jnp.float32)
        one = jnp.ones((CW,), jnp.float32)

        def block_body(t, bk, _):
            g = t * CH + bk * 16
            tm = lax.rem(t, 2)
            row0 = bk * 16
            bv = idxv[pl.ds(g, 16)]
            uniform = bv[0] == bv[15]

            @pl.when(uniform)
            def _():
                s = bv[0]
                sums = [rows[tm, row0, pl.ds(16 * j, 16)] for j in range(DV)]
                for r in range(1, 16):
                    for j in range(DV):
                        sums[j] = sums[j] + rows[tm, row0 + r, pl.ds(16 * j, 16)]
                for j in range(DV):
                    plsc.addupdate(acc.at[pl.ds(s * D + 16 * j, 16)], sums[j])
                plsc.addupdate(cntv.at[pl.ds(s * CW, CW)], sixteen)

            @pl.when(jnp.logical_not(uniform))
            def _():
                for r in range(16):
                    sr = bv[r]
                    for j in range(DV):
                        plsc.addupdate(
                            acc.at[pl.ds(sr * D + 16 * j, 16)],
                            rows[tm, row0 + r, pl.ds(16 * j, 16)])
                    plsc.addupdate(cntv.at[pl.ds(sr * CW, CW)], one)

            return 0

        def chunk_body(t, c):
            wait_t(t)

            @pl.when(t + 1 < nch)
            def _():
                issue(t + 1)

            return lax.fori_loop(0, NB, lambda bk, cc: block_body(t, bk, cc), c)

        issue(0)
        lax.fori_loop(0, nch, chunk_body, 0)

        pltpu.sync_copy(acc, sums_hbm.at[wid])
        pltpu.sync_copy(cntv, cnts_hbm.at[wid])

    return seg(xf, batch)


def _merge_kernel(s_ref, c_ref, o_ref):
    sums = jnp.sum(s_ref[...].reshape(NW, S, D), axis=0)
    cnt = jnp.sum(c_ref[...].reshape(NW, S, CW), axis=0)[:, 0:1]
    o_ref[...] = sums / jnp.maximum(cnt, 1.0)


def kernel(x, batch):
    sums, cnts = _sc_segment_sums(x, batch)
    return pl.pallas_call(
        _merge_kernel,
        out_shape=jax.ShapeDtypeStruct((S, D), jnp.float32),
    )(sums, cnts)


# speedup vs baseline: 2.7931x; 2.7931x over previous
import functools

import jax
import jax.numpy as jnp
from jax import lax
from jax.experimental import pallas as pl
from jax.experimental.pallas import tpu as pltpu
from jax.experimental.pallas import tpu_sc as plsc

N, D, S = 50000, 512, 64
NC, NS = 2, 16
NW = NC * NS
CH = 80
NB = CH // 16
TBC = 11
RW = CH * TBC
NSC = NW * RW
NTC = N - NSC
RT = 1040
GT = NTC // RT
DV = D // 16
CW = 16


def _sc_segment_sums(x, batch):
    mesh = plsc.VectorSubcoreMesh(core_axis_name="c", subcore_axis_name="s")

    @functools.partial(
        pl.kernel,
        mesh=mesh,
        out_type=[
            jax.ShapeDtypeStruct((NW, S * D), jnp.float32),
            jax.ShapeDtypeStruct((NW, S * CW), jnp.float32),
        ],
        scratch_types=[
            pltpu.VMEM((2, CH, D), jnp.float32),
            pltpu.VMEM((RW + 16,), jnp.int32),
            pltpu.VMEM((S * D,), jnp.float32),
            pltpu.VMEM((S * CW,), jnp.float32),
            pltpu.SemaphoreType.DMA,
            pltpu.SemaphoreType.DMA,
        ],
    )
    def seg(x_hbm, b_hbm, sums_hbm, cnts_hbm, rows, idxv, acc, cntv,
            sem0, sem1):
        cid = lax.axis_index("c")
        sid = lax.axis_index("s")
        wid = cid * NS + sid
        wbase = NTC + wid * RW

        zv = jnp.zeros((16,), jnp.float32)

        def zero_body(s, _):
            for j in range(DV):
                acc[pl.ds(s * D + 16 * j, 16)] = zv
            return 0

        lax.fori_loop(0, S, zero_body, 0)
        for k in range(S * CW // 16):
            cntv[pl.ds(16 * k, 16)] = zv

        pltpu.sync_copy(b_hbm.at[pl.ds(wbase, RW)], idxv.at[pl.ds(0, RW)])

        def issue(t):
            @pl.when(lax.rem(t, 2) == 0)
            def _():
                pltpu.async_copy(
                    x_hbm.at[pl.ds(wbase + t * CH, CH)], rows.at[0], sem0)

            @pl.when(lax.rem(t, 2) == 1)
            def _():
                pltpu.async_copy(
                    x_hbm.at[pl.ds(wbase + t * CH, CH)], rows.at[1], sem1)

        def wait_t(t):
            @pl.when(lax.rem(t, 2) == 0)
            def _():
                pltpu.make_async_copy(
                    x_hbm.at[pl.ds(wbase + t * CH, CH)], rows.at[0],
                    sem0).wait()

            @pl.when(lax.rem(t, 2) == 1)
            def _():
                pltpu.make_async_copy(
                    x_hbm.at[pl.ds(wbase + t * CH, CH)], rows.at[1],
                    sem1).wait()

        sixteen = jnp.full((CW,), 16.0, jnp.float32)
        one = jnp.ones((CW,), jnp.float32)

        def block_body(t, bk, _):
            g = t * CH + bk * 16
            tm = lax.rem(t, 2)
            row0 = bk * 16
            bv = idxv[pl.ds(g, 16)]
            uniform = bv[0] == bv[15]

            @pl.when(uniform)
            def _():
                s = bv[0]
                for j in range(DV):
                    a = rows[tm, row0, pl.ds(16 * j, 16)]
                    for r in range(1, 16):
                        a = a + rows[tm, row0 + r, pl.ds(16 * j, 16)]
                    plsc.addupdate(acc.at[pl.ds(s * D + 16 * j, 16)], a)
                plsc.addupdate(cntv.at[pl.ds(s * CW, CW)], sixteen)

            @pl.when(jnp.logical_not(uniform))
            def _():
                for r in range(16):
                    sr = bv[r]
                    for j in range(DV):
                        plsc.addupdate(
                            acc.at[pl.ds(sr * D + 16 * j, 16)],
                            rows[tm, row0 + r, pl.ds(16 * j, 16)])
                    plsc.addupdate(cntv.at[pl.ds(sr * CW, CW)], one)

            return 0

        def chunk_body(t, c):
            wait_t(t)

            @pl.when(t + 1 < TBC)
            def _():
                issue(t + 1)

            return lax.fori_loop(0, NB, lambda bk, cc: block_body(t, bk, cc), c)

        issue(0)
        lax.fori_loop(0, TBC, chunk_body, 0)

        pltpu.sync_copy(acc, sums_hbm.at[wid])
        pltpu.sync_copy(cntv, cnts_hbm.at[wid])

    return seg(x, batch)


def _tc_matmul_kernel(b_ref, x_ref, s_ref, c_ref):
    step = pl.program_id(0)

    @pl.when(step == 0)
    def _():
        s_ref[...] = jnp.zeros_like(s_ref)
        c_ref[...] = jnp.zeros_like(c_ref)

    ids = b_ref[0, 0, :]
    onehot = (lax.broadcasted_iota(jnp.int32, (S, RT), 0)
              == ids[None, :]).astype(jnp.float32)
    s_ref[...] += jax.lax.dot_general(
        onehot, x_ref[...], (((1,), (0,)), ((), ())),
        preferred_element_type=jnp.float32)
    c_ref[:, 0:1] += jnp.sum(onehot, axis=1, keepdims=True)


def _tc_segment_sums(x_tail, b_tail3):
    return pl.pallas_call(
        _tc_matmul_kernel,
        grid=(GT,),
        in_specs=[
            pl.BlockSpec((1, 1, RT), lambda t: (t, 0, 0)),
            pl.BlockSpec((RT, D), lambda t: (t, 0)),
        ],
        out_specs=[
            pl.BlockSpec((S, D), lambda t: (0, 0)),
            pl.BlockSpec((S, 128), lambda t: (0, 0)),
        ],
        out_shape=[
            jax.ShapeDtypeStruct((S, D), jnp.float32),
            jax.ShapeDtypeStruct((S, 128), jnp.float32),
        ],
        compiler_params=pltpu.CompilerParams(
            dimension_semantics=("arbitrary",)),
    )(b_tail3, x_tail)


def _merge_kernel(s_ref, c_ref, ts_ref, tc_ref, o_ref):
    sums = jnp.sum(s_ref[...].reshape(NW, S, D), axis=0) + ts_ref[...]
    cnt = (jnp.sum(c_ref[...].reshape(NW, S, CW), axis=0)[:, 0:1]
           + tc_ref[:, 0:1])
    o_ref[...] = sums / jnp.maximum(cnt, 1.0)


def kernel(x, batch):
    sc_sums, sc_cnts = _sc_segment_sums(x, batch)
    tc_sums, tc_cnts = _tc_segment_sums(
        x, batch[:NTC].reshape(GT, 1, RT))
    return pl.pallas_call(
        _merge_kernel,
        out_shape=jax.ShapeDtypeStruct((S, D), jnp.float32),
    )(sc_sums, sc_cnts, tc_sums, tc_cnts)


# speedup vs baseline: 3.2638x; 1.1685x over previous
import functools

import jax
import jax.numpy as jnp
from jax import lax
from jax.experimental import pallas as pl
from jax.experimental.pallas import tpu as pltpu
from jax.experimental.pallas import tpu_sc as plsc

N, D, S = 50000, 512, 64
NC, NS = 2, 16
NW = NC * NS
CH = 80
NB = CH // 16
TBC = 8
RW = CH * TBC
NSC = NW * RW
NTC = N - NSC
RT = 1640
GT = NTC // RT
DV = D // 16
CW = 16


def _sc_segment_sums(x, batch):
    mesh = plsc.VectorSubcoreMesh(core_axis_name="c", subcore_axis_name="s")

    @functools.partial(
        pl.kernel,
        mesh=mesh,
        out_type=[
            jax.ShapeDtypeStruct((NW, S * D), jnp.float32),
            jax.ShapeDtypeStruct((NW, S * CW), jnp.float32),
        ],
        scratch_types=[
            pltpu.VMEM((2, CH, D), jnp.float32),
            pltpu.VMEM((RW + 16,), jnp.int32),
            pltpu.VMEM((S * D,), jnp.float32),
            pltpu.VMEM((S * CW,), jnp.float32),
            pltpu.SemaphoreType.DMA,
            pltpu.SemaphoreType.DMA,
        ],
    )
    def seg(x_hbm, b_hbm, sums_hbm, cnts_hbm, rows, idxv, acc, cntv,
            sem0, sem1):
        cid = lax.axis_index("c")
        sid = lax.axis_index("s")
        wid = cid * NS + sid
        wbase = NTC + wid * RW

        zv = jnp.zeros((16,), jnp.float32)

        def zero_body(s, _):
            for j in range(DV):
                acc[pl.ds(s * D + 16 * j, 16)] = zv
            return 0

        lax.fori_loop(0, S, zero_body, 0)
        for k in range(S * CW // 16):
            cntv[pl.ds(16 * k, 16)] = zv

        pltpu.sync_copy(b_hbm.at[pl.ds(wbase, RW)], idxv.at[pl.ds(0, RW)])

        def issue(t):
            @pl.when(lax.rem(t, 2) == 0)
            def _():
                pltpu.async_copy(
                    x_hbm.at[pl.ds(wbase + t * CH, CH)], rows.at[0], sem0)

            @pl.when(lax.rem(t, 2) == 1)
            def _():
                pltpu.async_copy(
                    x_hbm.at[pl.ds(wbase + t * CH, CH)], rows.at[1], sem1)

        def wait_t(t):
            @pl.when(lax.rem(t, 2) == 0)
            def _():
                pltpu.make_async_copy(
                    x_hbm.at[pl.ds(wbase + t * CH, CH)], rows.at[0],
                    sem0).wait()

            @pl.when(lax.rem(t, 2) == 1)
            def _():
                pltpu.make_async_copy(
                    x_hbm.at[pl.ds(wbase + t * CH, CH)], rows.at[1],
                    sem1).wait()

        sixteen = jnp.full((CW,), 16.0, jnp.float32)
        one = jnp.ones((CW,), jnp.float32)

        def block_body(t, bk, _):
            g = t * CH + bk * 16
            tm = lax.rem(t, 2)
            row0 = bk * 16
            bv = idxv[pl.ds(g, 16)]
            uniform = bv[0] == bv[15]

            @pl.when(uniform)
            def _():
                s = bv[0]
                for j in range(DV):
                    a = rows[tm, row0, pl.ds(16 * j, 16)]
                    for r in range(1, 16):
                        a = a + rows[tm, row0 + r, pl.ds(16 * j, 16)]
                    plsc.addupdate(acc.at[pl.ds(s * D + 16 * j, 16)], a)
                plsc.addupdate(cntv.at[pl.ds(s * CW, CW)], sixteen)

            @pl.when(jnp.logical_not(uniform))
            def _():
                for r in range(16):
                    sr = bv[r]
                    for j in range(DV):
                        plsc.addupdate(
                            acc.at[pl.ds(sr * D + 16 * j, 16)],
                            rows[tm, row0 + r, pl.ds(16 * j, 16)])
                    plsc.addupdate(cntv.at[pl.ds(sr * CW, CW)], one)

            return 0

        def chunk_body(t, c):
            wait_t(t)

            @pl.when(t + 1 < TBC)
            def _():
                issue(t + 1)

            return lax.fori_loop(0, NB, lambda bk, cc: block_body(t, bk, cc), c)

        issue(0)
        lax.fori_loop(0, TBC, chunk_body, 0)

        pltpu.sync_copy(acc, sums_hbm.at[wid])
        pltpu.sync_copy(cntv, cnts_hbm.at[wid])

    return seg(x, batch)


def _tc_matmul_kernel(b_ref, x_ref, s_ref, c_ref):
    step = pl.program_id(0)

    @pl.when(step == 0)
    def _():
        s_ref[...] = jnp.zeros_like(s_ref)
        c_ref[...] = jnp.zeros_like(c_ref)

    ids = b_ref[0, 0, :]
    onehot = (lax.broadcasted_iota(jnp.int32, (S, RT), 0)
              == ids[None, :]).astype(jnp.float32)
    s_ref[...] += jax.lax.dot_general(
        onehot, x_ref[...], (((1,), (0,)), ((), ())),
        preferred_element_type=jnp.float32)
    c_ref[:, 0:1] += jnp.sum(onehot, axis=1, keepdims=True)


def _tc_segment_sums(x_tail, b_tail3):
    return pl.pallas_call(
        _tc_matmul_kernel,
        grid=(GT,),
        in_specs=[
            pl.BlockSpec((1, 1, RT), lambda t: (t, 0, 0)),
            pl.BlockSpec((RT, D), lambda t: (t, 0)),
        ],
        out_specs=[
            pl.BlockSpec((S, D), lambda t: (0, 0)),
            pl.BlockSpec((S, 128), lambda t: (0, 0)),
        ],
        out_shape=[
            jax.ShapeDtypeStruct((S, D), jnp.float32),
            jax.ShapeDtypeStruct((S, 128), jnp.float32),
        ],
        compiler_params=pltpu.CompilerParams(
            dimension_semantics=("arbitrary",)),
    )(b_tail3, x_tail)


def _merge_kernel(s_ref, c_ref, ts_ref, tc_ref, o_ref):
    sums = jnp.sum(s_ref[...].reshape(NW, S, D), axis=0) + ts_ref[...]
    cnt = (jnp.sum(c_ref[...].reshape(NW, S, CW), axis=0)[:, 0:1]
           + tc_ref[:, 0:1])
    o_ref[...] = sums / jnp.maximum(cnt, 1.0)


def kernel(x, batch):
    sc_sums, sc_cnts = _sc_segment_sums(x, batch)
    tc_sums, tc_cnts = _tc_segment_sums(
        x, batch[:NTC].reshape(GT, 1, RT))
    return pl.pallas_call(
        _merge_kernel,
        out_shape=jax.ShapeDtypeStruct((S, D), jnp.float32),
    )(sc_sums, sc_cnts, tc_sums, tc_cnts)


# speedup vs baseline: 3.3703x; 1.0326x over previous
import functools

import jax
import jax.numpy as jnp
from jax import lax
from jax.experimental import pallas as pl
from jax.experimental.pallas import tpu as pltpu
from jax.experimental.pallas import tpu_sc as plsc

N, D, S = 50000, 512, 64
NC, NS = 2, 16
NW = NC * NS
CH = 80
NB = CH // 16
TBC = 7
RW = CH * TBC
NSC = NW * RW
NTC = N - NSC
RT = 3208
GT = NTC // RT
DV = D // 16
CW = 16


def _sc_segment_sums(x, batch):
    mesh = plsc.VectorSubcoreMesh(core_axis_name="c", subcore_axis_name="s")

    @functools.partial(
        pl.kernel,
        mesh=mesh,
        out_type=[
            jax.ShapeDtypeStruct((NW, S * D), jnp.float32),
            jax.ShapeDtypeStruct((NW, S * CW), jnp.float32),
        ],
        scratch_types=[
            pltpu.VMEM((2, CH, D), jnp.float32),
            pltpu.VMEM((RW + 16,), jnp.int32),
            pltpu.VMEM((S * D,), jnp.float32),
            pltpu.VMEM((S * CW,), jnp.float32),
            pltpu.SemaphoreType.DMA,
            pltpu.SemaphoreType.DMA,
        ],
    )
    def seg(x_hbm, b_hbm, sums_hbm, cnts_hbm, rows, idxv, acc, cntv,
            sem0, sem1):
        cid = lax.axis_index("c")
        sid = lax.axis_index("s")
        wid = cid * NS + sid
        wbase = NTC + wid * RW

        zv = jnp.zeros((16,), jnp.float32)

        def zero_body(s, _):
            for j in range(DV):
                acc[pl.ds(s * D + 16 * j, 16)] = zv
            return 0

        lax.fori_loop(0, S, zero_body, 0)
        for k in range(S * CW // 16):
            cntv[pl.ds(16 * k, 16)] = zv

        pltpu.sync_copy(b_hbm.at[pl.ds(wbase, RW)], idxv.at[pl.ds(0, RW)])

        def issue(t):
            @pl.when(lax.rem(t, 2) == 0)
            def _():
                pltpu.async_copy(
                    x_hbm.at[pl.ds(wbase + t * CH, CH)], rows.at[0], sem0)

            @pl.when(lax.rem(t, 2) == 1)
            def _():
                pltpu.async_copy(
                    x_hbm.at[pl.ds(wbase + t * CH, CH)], rows.at[1], sem1)

        def wait_t(t):
            @pl.when(lax.rem(t, 2) == 0)
            def _():
                pltpu.make_async_copy(
                    x_hbm.at[pl.ds(wbase + t * CH, CH)], rows.at[0],
                    sem0).wait()

            @pl.when(lax.rem(t, 2) == 1)
            def _():
                pltpu.make_async_copy(
                    x_hbm.at[pl.ds(wbase + t * CH, CH)], rows.at[1],
                    sem1).wait()

        sixteen = jnp.full((CW,), 16.0, jnp.float32)
        one = jnp.ones((CW,), jnp.float32)

        def block_body(t, bk, _):
            g = t * CH + bk * 16
            tm = lax.rem(t, 2)
            row0 = bk * 16
            bv = idxv[pl.ds(g, 16)]
            uniform = bv[0] == bv[15]

            @pl.when(uniform)
            def _():
                s = bv[0]
                for j in range(DV):
                    a = rows[tm, row0, pl.ds(16 * j, 16)]
                    for r in range(1, 16):
                        a = a + rows[tm, row0 + r, pl.ds(16 * j, 16)]
                    plsc.addupdate(acc.at[pl.ds(s * D + 16 * j, 16)], a)
                plsc.addupdate(cntv.at[pl.ds(s * CW, CW)], sixteen)

            @pl.when(jnp.logical_not(uniform))
            def _():
                for r in range(16):
                    sr = bv[r]
                    for j in range(DV):
                        plsc.addupdate(
                            acc.at[pl.ds(sr * D + 16 * j, 16)],
                            rows[tm, row0 + r, pl.ds(16 * j, 16)])
                    plsc.addupdate(cntv.at[pl.ds(sr * CW, CW)], one)

            return 0

        def chunk_body(t, c):
            wait_t(t)

            @pl.when(t + 1 < TBC)
            def _():
                issue(t + 1)

            return lax.fori_loop(0, NB, lambda bk, cc: block_body(t, bk, cc), c)

        issue(0)
        lax.fori_loop(0, TBC, chunk_body, 0)

        pltpu.sync_copy(acc, sums_hbm.at[wid])
        pltpu.sync_copy(cntv, cnts_hbm.at[wid])

    return seg(x, batch)


def _tc_matmul_kernel(b_ref, x_ref, s_ref, c_ref):
    step = pl.program_id(0)

    @pl.when(step == 0)
    def _():
        s_ref[...] = jnp.zeros_like(s_ref)
        c_ref[...] = jnp.zeros_like(c_ref)

    ids = b_ref[0, 0, :]
    onehot = (lax.broadcasted_iota(jnp.int32, (S, RT), 0)
              == ids[None, :]).astype(jnp.float32)
    s_ref[...] += jax.lax.dot_general(
        onehot, x_ref[...], (((1,), (0,)), ((), ())),
        preferred_element_type=jnp.float32)
    c_ref[:, 0:1] += jnp.sum(onehot, axis=1, keepdims=True)


def _tc_segment_sums(x_tail, b_tail3):
    return pl.pallas_call(
        _tc_matmul_kernel,
        grid=(GT,),
        in_specs=[
            pl.BlockSpec((1, 1, RT), lambda t: (t, 0, 0)),
            pl.BlockSpec((RT, D), lambda t: (t, 0)),
        ],
        out_specs=[
            pl.BlockSpec((S, D), lambda t: (0, 0)),
            pl.BlockSpec((S, 128), lambda t: (0, 0)),
        ],
        out_shape=[
            jax.ShapeDtypeStruct((S, D), jnp.float32),
            jax.ShapeDtypeStruct((S, 128), jnp.float32),
        ],
        compiler_params=pltpu.CompilerParams(
            dimension_semantics=("arbitrary",)),
    )(b_tail3, x_tail)


def _merge_kernel(s_ref, c_ref, ts_ref, tc_ref, o_ref):
    sums = jnp.sum(s_ref[...].reshape(NW, S, D), axis=0) + ts_ref[...]
    cnt = (jnp.sum(c_ref[...].reshape(NW, S, CW), axis=0)[:, 0:1]
           + tc_ref[:, 0:1])
    o_ref[...] = sums / jnp.maximum(cnt, 1.0)


def kernel(x, batch):
    sc_sums, sc_cnts = _sc_segment_sums(x, batch)
    tc_sums, tc_cnts = _tc_segment_sums(
        x, batch[:NTC].reshape(GT, 1, RT))
    return pl.pallas_call(
        _merge_kernel,
        out_shape=jax.ShapeDtypeStruct((S, D), jnp.float32),
    )(sc_sums, sc_cnts, tc_sums, tc_cnts)


# speedup vs baseline: 3.5758x; 1.0610x over previous
import functools

import jax
import jax.numpy as jnp
from jax import lax
from jax.experimental import pallas as pl
from jax.experimental.pallas import tpu as pltpu
from jax.experimental.pallas import tpu_sc as plsc

N, D, S = 50000, 512, 64
NC, NS = 2, 16
NW = NC * NS
CH = 80
NB = CH // 16
TBC = 6
RW = CH * TBC
NSC = NW * RW
NTC = N - NSC
RT = 3464
GT = NTC // RT
DV = D // 16
CW = 16


def _sc_segment_sums(x, batch):
    mesh = plsc.VectorSubcoreMesh(core_axis_name="c", subcore_axis_name="s")

    @functools.partial(
        pl.kernel,
        mesh=mesh,
        out_type=[
            jax.ShapeDtypeStruct((NW, S * D), jnp.float32),
            jax.ShapeDtypeStruct((NW, S * CW), jnp.float32),
        ],
        scratch_types=[
            pltpu.VMEM((2, CH, D), jnp.float32),
            pltpu.VMEM((RW + 16,), jnp.int32),
            pltpu.VMEM((S * D,), jnp.float32),
            pltpu.VMEM((S * CW,), jnp.float32),
            pltpu.SemaphoreType.DMA,
            pltpu.SemaphoreType.DMA,
        ],
    )
    def seg(x_hbm, b_hbm, sums_hbm, cnts_hbm, rows, idxv, acc, cntv,
            sem0, sem1):
        cid = lax.axis_index("c")
        sid = lax.axis_index("s")
        wid = cid * NS + sid
        wbase = NTC + wid * RW

        zv = jnp.zeros((16,), jnp.float32)

        def zero_body(s, _):
            for j in range(DV):
                acc[pl.ds(s * D + 16 * j, 16)] = zv
            return 0

        lax.fori_loop(0, S, zero_body, 0)
        for k in range(S * CW // 16):
            cntv[pl.ds(16 * k, 16)] = zv

        pltpu.sync_copy(b_hbm.at[pl.ds(wbase, RW)], idxv.at[pl.ds(0, RW)])

        def issue(t):
            @pl.when(lax.rem(t, 2) == 0)
            def _():
                pltpu.async_copy(
                    x_hbm.at[pl.ds(wbase + t * CH, CH)], rows.at[0], sem0)

            @pl.when(lax.rem(t, 2) == 1)
            def _():
                pltpu.async_copy(
                    x_hbm.at[pl.ds(wbase + t * CH, CH)], rows.at[1], sem1)

        def wait_t(t):
            @pl.when(lax.rem(t, 2) == 0)
            def _():
                pltpu.make_async_copy(
                    x_hbm.at[pl.ds(wbase + t * CH, CH)], rows.at[0],
                    sem0).wait()

            @pl.when(lax.rem(t, 2) == 1)
            def _():
                pltpu.make_async_copy(
                    x_hbm.at[pl.ds(wbase + t * CH, CH)], rows.at[1],
                    sem1).wait()

        sixteen = jnp.full((CW,), 16.0, jnp.float32)
        one = jnp.ones((CW,), jnp.float32)

        def block_body(t, bk, _):
            g = t * CH + bk * 16
            tm = lax.rem(t, 2)
            row0 = bk * 16
            bv = idxv[pl.ds(g, 16)]
            uniform = bv[0] == bv[15]

            @pl.when(uniform)
            def _():
                s = bv[0]
                for j in range(DV):
                    a = rows[tm, row0, pl.ds(16 * j, 16)]
                    for r in range(1, 16):
                        a = a + rows[tm, row0 + r, pl.ds(16 * j, 16)]
                    plsc.addupdate(acc.at[pl.ds(s * D + 16 * j, 16)], a)
                plsc.addupdate(cntv.at[pl.ds(s * CW, CW)], sixteen)

            @pl.when(jnp.logical_not(uniform))
            def _():
                for r in range(16):
                    sr = bv[r]
                    for j in range(DV):
                        plsc.addupdate(
                            acc.at[pl.ds(sr * D + 16 * j, 16)],
                            rows[tm, row0 + r, pl.ds(16 * j, 16)])
                    plsc.addupdate(cntv.at[pl.ds(sr * CW, CW)], one)

            return 0

        def chunk_body(t, c):
            wait_t(t)

            @pl.when(t + 1 < TBC)
            def _():
                issue(t + 1)

            return lax.fori_loop(0, NB, lambda bk, cc: block_body(t, bk, cc), c)

        issue(0)
        lax.fori_loop(0, TBC, chunk_body, 0)

        pltpu.sync_copy(acc, sums_hbm.at[wid])
        pltpu.sync_copy(cntv, cnts_hbm.at[wid])

    return seg(x, batch)


def _tc_matmul_kernel(b_ref, x_ref, s_ref, c_ref):
    step = pl.program_id(0)

    @pl.when(step == 0)
    def _():
        s_ref[...] = jnp.zeros_like(s_ref)
        c_ref[...] = jnp.zeros_like(c_ref)

    ids = b_ref[0, 0, :]
    onehot = (lax.broadcasted_iota(jnp.int32, (S, RT), 0)
              == ids[None, :]).astype(jnp.float32)
    s_ref[...] += jax.lax.dot_general(
        onehot, x_ref[...], (((1,), (0,)), ((), ())),
        preferred_element_type=jnp.float32)
    c_ref[:, 0:1] += jnp.sum(onehot, axis=1, keepdims=True)


def _tc_segment_sums(x_tail, b_tail3):
    return pl.pallas_call(
        _tc_matmul_kernel,
        grid=(GT,),
        in_specs=[
            pl.BlockSpec((1, 1, RT), lambda t: (t, 0, 0)),
            pl.BlockSpec((RT, D), lambda t: (t, 0)),
        ],
        out_specs=[
            pl.BlockSpec((S, D), lambda t: (0, 0)),
            pl.BlockSpec((S, 128), lambda t: (0, 0)),
        ],
        out_shape=[
            jax.ShapeDtypeStruct((S, D), jnp.float32),
            jax.ShapeDtypeStruct((S, 128), jnp.float32),
        ],
        compiler_params=pltpu.CompilerParams(
            dimension_semantics=("arbitrary",)),
    )(b_tail3, x_tail)


def _merge_kernel(s_ref, c_ref, ts_ref, tc_ref, o_ref):
    sums = jnp.sum(s_ref[...].reshape(NW, S, D), axis=0) + ts_ref[...]
    cnt = (jnp.sum(c_ref[...].reshape(NW, S, CW), axis=0)[:, 0:1]
           + tc_ref[:, 0:1])
    o_ref[...] = sums / jnp.maximum(cnt, 1.0)


def kernel(x, batch):
    sc_sums, sc_cnts = _sc_segment_sums(x, batch)
    tc_sums, tc_cnts = _tc_segment_sums(
        x, batch[:NTC].reshape(GT, 1, RT))
    return pl.pallas_call(
        _merge_kernel,
        out_shape=jax.ShapeDtypeStruct((S, D), jnp.float32),
    )(sc_sums, sc_cnts, tc_sums, tc_cnts)


# speedup vs baseline: 3.7762x; 1.0561x over previous
import functools

import jax
import jax.numpy as jnp
from jax import lax
from jax.experimental import pallas as pl
from jax.experimental.pallas import tpu as pltpu
from jax.experimental.pallas import tpu_sc as plsc

N, D, S = 50000, 512, 64
NC, NS = 2, 16
NW = NC * NS
CH = 80
NB = CH // 16
TBC = 5
RW = CH * TBC
NSC = NW * RW
NTC = N - NSC
RT = 3720
GT = NTC // RT
DV = D // 16
CW = 16


def _sc_segment_sums(x, batch):
    mesh = plsc.VectorSubcoreMesh(core_axis_name="c", subcore_axis_name="s")

    @functools.partial(
        pl.kernel,
        mesh=mesh,
        out_type=[
            jax.ShapeDtypeStruct((NW, S * D), jnp.float32),
            jax.ShapeDtypeStruct((NW, S * CW), jnp.float32),
        ],
        scratch_types=[
            pltpu.VMEM((2, CH, D), jnp.float32),
            pltpu.VMEM((RW + 16,), jnp.int32),
            pltpu.VMEM((S * D,), jnp.float32),
            pltpu.VMEM((S * CW,), jnp.float32),
            pltpu.SemaphoreType.DMA,
            pltpu.SemaphoreType.DMA,
        ],
    )
    def seg(x_hbm, b_hbm, sums_hbm, cnts_hbm, rows, idxv, acc, cntv,
            sem0, sem1):
        cid = lax.axis_index("c")
        sid = lax.axis_index("s")
        wid = cid * NS + sid
        wbase = NTC + wid * RW

        zv = jnp.zeros((16,), jnp.float32)

        def zero_body(s, _):
            for j in range(DV):
                acc[pl.ds(s * D + 16 * j, 16)] = zv
            return 0

        lax.fori_loop(0, S, zero_body, 0)
        for k in range(S * CW // 16):
            cntv[pl.ds(16 * k, 16)] = zv

        pltpu.sync_copy(b_hbm.at[pl.ds(wbase, RW)], idxv.at[pl.ds(0, RW)])

        def issue(t):
            @pl.when(lax.rem(t, 2) == 0)
            def _():
                pltpu.async_copy(
                    x_hbm.at[pl.ds(wbase + t * CH, CH)], rows.at[0], sem0)

            @pl.when(lax.rem(t, 2) == 1)
            def _():
                pltpu.async_copy(
                    x_hbm.at[pl.ds(wbase + t * CH, CH)], rows.at[1], sem1)

        def wait_t(t):
            @pl.when(lax.rem(t, 2) == 0)
            def _():
                pltpu.make_async_copy(
                    x_hbm.at[pl.ds(wbase + t * CH, CH)], rows.at[0],
                    sem0).wait()

            @pl.when(lax.rem(t, 2) == 1)
            def _():
                pltpu.make_async_copy(
                    x_hbm.at[pl.ds(wbase + t * CH, CH)], rows.at[1],
                    sem1).wait()

        sixteen = jnp.full((CW,), 16.0, jnp.float32)
        one = jnp.ones((CW,), jnp.float32)

        def block_body(t, bk, _):
            g = t * CH + bk * 16
            tm = lax.rem(t, 2)
            row0 = bk * 16
            bv = idxv[pl.ds(g, 16)]
            uniform = bv[0] == bv[15]

            @pl.when(uniform)
            def _():
                s = bv[0]
                for j in range(DV):
                    a = rows[tm, row0, pl.ds(16 * j, 16)]
                    for r in range(1, 16):
                        a = a + rows[tm, row0 + r, pl.ds(16 * j, 16)]
                    plsc.addupdate(acc.at[pl.ds(s * D + 16 * j, 16)], a)
                plsc.addupdate(cntv.at[pl.ds(s * CW, CW)], sixteen)

            @pl.when(jnp.logical_not(uniform))
            def _():
                for r in range(16):
                    sr = bv[r]
                    for j in range(DV):
                        plsc.addupdate(
                            acc.at[pl.ds(sr * D + 16 * j, 16)],
                            rows[tm, row0 + r, pl.ds(16 * j, 16)])
                    plsc.addupdate(cntv.at[pl.ds(sr * CW, CW)], one)

            return 0

        def chunk_body(t, c):
            wait_t(t)

            @pl.when(t + 1 < TBC)
            def _():
                issue(t + 1)

            return lax.fori_loop(0, NB, lambda bk, cc: block_body(t, bk, cc), c)

        issue(0)
        lax.fori_loop(0, TBC, chunk_body, 0)

        pltpu.sync_copy(acc, sums_hbm.at[wid])
        pltpu.sync_copy(cntv, cnts_hbm.at[wid])

    return seg(x, batch)


def _tc_matmul_kernel(b_ref, x_ref, s_ref, c_ref):
    step = pl.program_id(0)

    @pl.when(step == 0)
    def _():
        s_ref[...] = jnp.zeros_like(s_ref)
        c_ref[...] = jnp.zeros_like(c_ref)

    ids = b_ref[0, 0, :]
    onehot = (lax.broadcasted_iota(jnp.int32, (S, RT), 0)
              == ids[None, :]).astype(jnp.float32)
    s_ref[...] += jax.lax.dot_general(
        onehot, x_ref[...], (((1,), (0,)), ((), ())),
        preferred_element_type=jnp.float32)
    c_ref[:, 0:1] += jnp.sum(onehot, axis=1, keepdims=True)


def _tc_segment_sums(x_tail, b_tail3):
    return pl.pallas_call(
        _tc_matmul_kernel,
        grid=(GT,),
        in_specs=[
            pl.BlockSpec((1, 1, RT), lambda t: (t, 0, 0)),
            pl.BlockSpec((RT, D), lambda t: (t, 0)),
        ],
        out_specs=[
            pl.BlockSpec((S, D), lambda t: (0, 0)),
            pl.BlockSpec((S, 128), lambda t: (0, 0)),
        ],
        out_shape=[
            jax.ShapeDtypeStruct((S, D), jnp.float32),
            jax.ShapeDtypeStruct((S, 128), jnp.float32),
        ],
        compiler_params=pltpu.CompilerParams(
            dimension_semantics=("arbitrary",)),
    )(b_tail3, x_tail)


def _merge_kernel(s_ref, c_ref, ts_ref, tc_ref, o_ref):
    sums = jnp.sum(s_ref[...].reshape(NW, S, D), axis=0) + ts_ref[...]
    cnt = (jnp.sum(c_ref[...].reshape(NW, S, CW), axis=0)[:, 0:1]
           + tc_ref[:, 0:1])
    o_ref[...] = sums / jnp.maximum(cnt, 1.0)


def kernel(x, batch):
    sc_sums, sc_cnts = _sc_segment_sums(x, batch)
    tc_sums, tc_cnts = _tc_segment_sums(
        x, batch[:NTC].reshape(GT, 1, RT))
    return pl.pallas_call(
        _merge_kernel,
        out_shape=jax.ShapeDtypeStruct((S, D), jnp.float32),
    )(sc_sums, sc_cnts, tc_sums, tc_cnts)


# speedup vs baseline: 3.9078x; 1.0349x over previous
import functools

import jax
import jax.numpy as jnp
from jax import lax
from jax.experimental import pallas as pl
from jax.experimental.pallas import tpu as pltpu
from jax.experimental.pallas import tpu_sc as plsc

N, D, S = 50000, 512, 64
NC, NS = 2, 16
NW = NC * NS
CH = 80
NB = CH // 16
TBC = 4
RW = CH * TBC
NSC = NW * RW
NTC = N - NSC
RT = 3976
GT = NTC // RT
DV = D // 16
CW = 16


def _sc_segment_sums(x, batch):
    mesh = plsc.VectorSubcoreMesh(core_axis_name="c", subcore_axis_name="s")

    @functools.partial(
        pl.kernel,
        mesh=mesh,
        out_type=[
            jax.ShapeDtypeStruct((NW, S * D), jnp.float32),
            jax.ShapeDtypeStruct((NW, S * CW), jnp.float32),
        ],
        scratch_types=[
            pltpu.VMEM((2, CH, D), jnp.float32),
            pltpu.VMEM((RW + 16,), jnp.int32),
            pltpu.VMEM((S * D,), jnp.float32),
            pltpu.VMEM((S * CW,), jnp.float32),
            pltpu.SemaphoreType.DMA,
            pltpu.SemaphoreType.DMA,
        ],
    )
    def seg(x_hbm, b_hbm, sums_hbm, cnts_hbm, rows, idxv, acc, cntv,
            sem0, sem1):
        cid = lax.axis_index("c")
        sid = lax.axis_index("s")
        wid = cid * NS + sid
        wbase = NTC + wid * RW

        zv = jnp.zeros((16,), jnp.float32)

        def zero_body(s, _):
            for j in range(DV):
                acc[pl.ds(s * D + 16 * j, 16)] = zv
            return 0

        lax.fori_loop(0, S, zero_body, 0)
        for k in range(S * CW // 16):
            cntv[pl.ds(16 * k, 16)] = zv

        pltpu.sync_copy(b_hbm.at[pl.ds(wbase, RW)], idxv.at[pl.ds(0, RW)])

        def issue(t):
            @pl.when(lax.rem(t, 2) == 0)
            def _():
                pltpu.async_copy(
                    x_hbm.at[pl.ds(wbase + t * CH, CH)], rows.at[0], sem0)

            @pl.when(lax.rem(t, 2) == 1)
            def _():
                pltpu.async_copy(
                    x_hbm.at[pl.ds(wbase + t * CH, CH)], rows.at[1], sem1)

        def wait_t(t):
            @pl.when(lax.rem(t, 2) == 0)
            def _():
                pltpu.make_async_copy(
                    x_hbm.at[pl.ds(wbase + t * CH, CH)], rows.at[0],
                    sem0).wait()

            @pl.when(lax.rem(t, 2) == 1)
            def _():
                pltpu.make_async_copy(
                    x_hbm.at[pl.ds(wbase + t * CH, CH)], rows.at[1],
                    sem1).wait()

        sixteen = jnp.full((CW,), 16.0, jnp.float32)
        one = jnp.ones((CW,), jnp.float32)

        def block_body(t, bk, _):
            g = t * CH + bk * 16
            tm = lax.rem(t, 2)
            row0 = bk * 16
            bv = idxv[pl.ds(g, 16)]
            uniform = bv[0] == bv[15]

            @pl.when(uniform)
            def _():
                s = bv[0]
                for j in range(DV):
                    a = rows[tm, row0, pl.ds(16 * j, 16)]
                    for r in range(1, 16):
                        a = a + rows[tm, row0 + r, pl.ds(16 * j, 16)]
                    plsc.addupdate(acc.at[pl.ds(s * D + 16 * j, 16)], a)
                plsc.addupdate(cntv.at[pl.ds(s * CW, CW)], sixteen)

            @pl.when(jnp.logical_not(uniform))
            def _():
                for r in range(16):
                    sr = bv[r]
                    for j in range(DV):
                        plsc.addupdate(
                            acc.at[pl.ds(sr * D + 16 * j, 16)],
                            rows[tm, row0 + r, pl.ds(16 * j, 16)])
                    plsc.addupdate(cntv.at[pl.ds(sr * CW, CW)], one)

            return 0

        def chunk_body(t, c):
            wait_t(t)

            @pl.when(t + 1 < TBC)
            def _():
                issue(t + 1)

            return lax.fori_loop(0, NB, lambda bk, cc: block_body(t, bk, cc), c)

        issue(0)
        lax.fori_loop(0, TBC, chunk_body, 0)

        pltpu.sync_copy(acc, sums_hbm.at[wid])
        pltpu.sync_copy(cntv, cnts_hbm.at[wid])

    return seg(x, batch)


def _tc_matmul_kernel(b_ref, x_ref, s_ref, c_ref):
    step = pl.program_id(0)

    @pl.when(step == 0)
    def _():
        s_ref[...] = jnp.zeros_like(s_ref)
        c_ref[...] = jnp.zeros_like(c_ref)

    ids = b_ref[0, 0, :]
    onehot = (lax.broadcasted_iota(jnp.int32, (S, RT), 0)
              == ids[None, :]).astype(jnp.float32)
    s_ref[...] += jax.lax.dot_general(
        onehot, x_ref[...], (((1,), (0,)), ((), ())),
        preferred_element_type=jnp.float32)
    c_ref[:, 0:1] += jnp.sum(onehot, axis=1, keepdims=True)


def _tc_segment_sums(x_tail, b_tail3):
    return pl.pallas_call(
        _tc_matmul_kernel,
        grid=(GT,),
        in_specs=[
            pl.BlockSpec((1, 1, RT), lambda t: (t, 0, 0)),
            pl.BlockSpec((RT, D), lambda t: (t, 0)),
        ],
        out_specs=[
            pl.BlockSpec((S, D), lambda t: (0, 0)),
            pl.BlockSpec((S, 128), lambda t: (0, 0)),
        ],
        out_shape=[
            jax.ShapeDtypeStruct((S, D), jnp.float32),
            jax.ShapeDtypeStruct((S, 128), jnp.float32),
        ],
        compiler_params=pltpu.CompilerParams(
            dimension_semantics=("arbitrary",)),
    )(b_tail3, x_tail)


def _merge_kernel(s_ref, c_ref, ts_ref, tc_ref, o_ref):
    sums = jnp.sum(s_ref[...].reshape(NW, S, D), axis=0) + ts_ref[...]
    cnt = (jnp.sum(c_ref[...].reshape(NW, S, CW), axis=0)[:, 0:1]
           + tc_ref[:, 0:1])
    o_ref[...] = sums / jnp.maximum(cnt, 1.0)


def kernel(x, batch):
    sc_sums, sc_cnts = _sc_segment_sums(x, batch)
    tc_sums, tc_cnts = _tc_segment_sums(
        x, batch[:NTC].reshape(GT, 1, RT))
    return pl.pallas_call(
        _merge_kernel,
        out_shape=jax.ShapeDtypeStruct((S, D), jnp.float32),
    )(sc_sums, sc_cnts, tc_sums, tc_cnts)


# speedup vs baseline: 3.9198x; 1.0030x over previous
import functools

import jax
import jax.numpy as jnp
from jax import lax
from jax.experimental import pallas as pl
from jax.experimental.pallas import tpu as pltpu
from jax.experimental.pallas import tpu_sc as plsc

N, D, S = 50000, 512, 64
NC, NS = 2, 16
NW = NC * NS
CH = 80
NB = CH // 16
TBC = 3
RW = CH * TBC
NSC = NW * RW
NTC = N - NSC
RT = 4232
GT = NTC // RT
DV = D // 16
CW = 16


def _sc_segment_sums(x, batch):
    mesh = plsc.VectorSubcoreMesh(core_axis_name="c", subcore_axis_name="s")

    @functools.partial(
        pl.kernel,
        mesh=mesh,
        out_type=[
            jax.ShapeDtypeStruct((NW, S * D), jnp.float32),
            jax.ShapeDtypeStruct((NW, S * CW), jnp.float32),
        ],
        scratch_types=[
            pltpu.VMEM((2, CH, D), jnp.float32),
            pltpu.VMEM((RW + 16,), jnp.int32),
            pltpu.VMEM((S * D,), jnp.float32),
            pltpu.VMEM((S * CW,), jnp.float32),
            pltpu.SemaphoreType.DMA,
            pltpu.SemaphoreType.DMA,
        ],
    )
    def seg(x_hbm, b_hbm, sums_hbm, cnts_hbm, rows, idxv, acc, cntv,
            sem0, sem1):
        cid = lax.axis_index("c")
        sid = lax.axis_index("s")
        wid = cid * NS + sid
        wbase = NTC + wid * RW

        zv = jnp.zeros((16,), jnp.float32)

        def zero_body(s, _):
            for j in range(DV):
                acc[pl.ds(s * D + 16 * j, 16)] = zv
            return 0

        lax.fori_loop(0, S, zero_body, 0)
        for k in range(S * CW // 16):
            cntv[pl.ds(16 * k, 16)] = zv

        pltpu.sync_copy(b_hbm.at[pl.ds(wbase, RW)], idxv.at[pl.ds(0, RW)])

        def issue(t):
            @pl.when(lax.rem(t, 2) == 0)
            def _():
                pltpu.async_copy(
                    x_hbm.at[pl.ds(wbase + t * CH, CH)], rows.at[0], sem0)

            @pl.when(lax.rem(t, 2) == 1)
            def _():
                pltpu.async_copy(
                    x_hbm.at[pl.ds(wbase + t * CH, CH)], rows.at[1], sem1)

        def wait_t(t):
            @pl.when(lax.rem(t, 2) == 0)
            def _():
                pltpu.make_async_copy(
                    x_hbm.at[pl.ds(wbase + t * CH, CH)], rows.at[0],
                    sem0).wait()

            @pl.when(lax.rem(t, 2) == 1)
            def _():
                pltpu.make_async_copy(
                    x_hbm.at[pl.ds(wbase + t * CH, CH)], rows.at[1],
                    sem1).wait()

        sixteen = jnp.full((CW,), 16.0, jnp.float32)
        one = jnp.ones((CW,), jnp.float32)

        def block_body(t, bk, _):
            g = t * CH + bk * 16
            tm = lax.rem(t, 2)
            row0 = bk * 16
            bv = idxv[pl.ds(g, 16)]
            uniform = bv[0] == bv[15]

            @pl.when(uniform)
            def _():
                s = bv[0]
                for j in range(DV):
                    a = rows[tm, row0, pl.ds(16 * j, 16)]
                    for r in range(1, 16):
                        a = a + rows[tm, row0 + r, pl.ds(16 * j, 16)]
                    plsc.addupdate(acc.at[pl.ds(s * D + 16 * j, 16)], a)
                plsc.addupdate(cntv.at[pl.ds(s * CW, CW)], sixteen)

            @pl.when(jnp.logical_not(uniform))
            def _():
                for r in range(16):
                    sr = bv[r]
                    for j in range(DV):
                        plsc.addupdate(
                            acc.at[pl.ds(sr * D + 16 * j, 16)],
                            rows[tm, row0 + r, pl.ds(16 * j, 16)])
                    plsc.addupdate(cntv.at[pl.ds(sr * CW, CW)], one)

            return 0

        def chunk_body(t, c):
            wait_t(t)

            @pl.when(t + 1 < TBC)
            def _():
                issue(t + 1)

            return lax.fori_loop(0, NB, lambda bk, cc: block_body(t, bk, cc), c)

        issue(0)
        lax.fori_loop(0, TBC, chunk_body, 0)

        pltpu.sync_copy(acc, sums_hbm.at[wid])
        pltpu.sync_copy(cntv, cnts_hbm.at[wid])

    return seg(x, batch)


def _tc_matmul_kernel(b_ref, x_ref, s_ref, c_ref):
    step = pl.program_id(0)

    @pl.when(step == 0)
    def _():
        s_ref[...] = jnp.zeros_like(s_ref)
        c_ref[...] = jnp.zeros_like(c_ref)

    ids = b_ref[0, 0, :]
    onehot = (lax.broadcasted_iota(jnp.int32, (S, RT), 0)
              == ids[None, :]).astype(jnp.float32)
    s_ref[...] += jax.lax.dot_general(
        onehot, x_ref[...], (((1,), (0,)), ((), ())),
        preferred_element_type=jnp.float32)
    c_ref[:, 0:1] += jnp.sum(onehot, axis=1, keepdims=True)


def _tc_segment_sums(x_tail, b_tail3):
    return pl.pallas_call(
        _tc_matmul_kernel,
        grid=(GT,),
        in_specs=[
            pl.BlockSpec((1, 1, RT), lambda t: (t, 0, 0)),
            pl.BlockSpec((RT, D), lambda t: (t, 0)),
        ],
        out_specs=[
            pl.BlockSpec((S, D), lambda t: (0, 0)),
            pl.BlockSpec((S, 128), lambda t: (0, 0)),
        ],
        out_shape=[
            jax.ShapeDtypeStruct((S, D), jnp.float32),
            jax.ShapeDtypeStruct((S, 128), jnp.float32),
        ],
        compiler_params=pltpu.CompilerParams(
            dimension_semantics=("arbitrary",)),
    )(b_tail3, x_tail)


def _merge_kernel(s_ref, c_ref, ts_ref, tc_ref, o_ref):
    sums = jnp.sum(s_ref[...].reshape(NW, S, D), axis=0) + ts_ref[...]
    cnt = (jnp.sum(c_ref[...].reshape(NW, S, CW), axis=0)[:, 0:1]
           + tc_ref[:, 0:1])
    o_ref[...] = sums / jnp.maximum(cnt, 1.0)


def kernel(x, batch):
    sc_sums, sc_cnts = _sc_segment_sums(x, batch)
    tc_sums, tc_cnts = _tc_segment_sums(
        x, batch[:NTC].reshape(GT, 1, RT))
    return pl.pallas_call(
        _merge_kernel,
        out_shape=jax.ShapeDtypeStruct((S, D), jnp.float32),
    )(sc_sums, sc_cnts, tc_sums, tc_cnts)
